# Initial kernel scaffold; baseline (speedup 1.0000x reference)
import jax
import jax.numpy as jnp
from jax.experimental import pallas as pl


def kernel(features, edge_index, W, b):
    raise NotImplementedError("write your pallas kernel here")


# speedup vs baseline: 6.5017x; 6.5017x over previous
import functools

import jax
import jax.numpy as jnp
from jax import lax
from jax.experimental import pallas as pl
from jax.experimental.pallas import tpu as pltpu
from jax.experimental.pallas import tpu_sc as plsc

_N = 10000
_E = 320000
_D = 128
_NC = 2
_NS = 16
_NH = _N // _NC
_EPT = _E // _NS
_CH = 80
_NCHUNK = _EPT // _CH
_NP = 5248
_RPT = _NP // _NS

_mesh = plsc.VectorSubcoreMesh(core_axis_name="c", subcore_axis_name="s")


@functools.partial(
    pl.kernel,
    out_type=jax.ShapeDtypeStruct((_NC, _NP, _D), jnp.float32),
    mesh=_mesh,
    scratch_types=[
        pltpu.VMEM((_NCHUNK, _CH), jnp.int32),
        pltpu.VMEM((_NCHUNK, _CH), jnp.int32),
        pltpu.VMEM((_CH, _D), jnp.float32),
        pltpu.VMEM((_CH, _D), jnp.float32),
        pltpu.VMEM_SHARED((_NP, _D), jnp.float32),
        pltpu.SemaphoreType.DMA,
        pltpu.SemaphoreType.DMA,
    ],
)
def _sc_gather_scatter(feat_hbm, src_hbm, dst_hbm, zeros_hbm, out_hbm,
                       src_v, dst_v, buf0, buf1, h_sh, sem0, sem1):
    c = lax.axis_index("c")
    s = lax.axis_index("s")
    r0 = s * _RPT

    pltpu.sync_copy(zeros_hbm.at[pl.ds(r0, _RPT)],
                    h_sh.at[pl.ds(r0, _RPT)])
    pltpu.sync_copy(src_hbm.at[s], src_v)
    pltpu.sync_copy(dst_hbm.at[c, s], dst_v)
    plsc.subcore_barrier()

    def wait_gather(buf, sem):
        pltpu.make_async_copy(feat_hbm.at[pl.ds(0, _CH)], buf, sem).wait()

    pltpu.async_copy(feat_hbm.at[src_v.at[0]], buf0, sem0)
    pltpu.async_copy(feat_hbm.at[src_v.at[1]], buf1, sem1)

    def step(k, _):
        j = 2 * k
        wait_gather(buf0, sem0)
        pltpu.sync_copy(buf0, h_sh.at[dst_v.at[j]], add=True)
        pltpu.async_copy(feat_hbm.at[src_v.at[j + 2]], buf0, sem0)
        wait_gather(buf1, sem1)
        pltpu.sync_copy(buf1, h_sh.at[dst_v.at[j + 1]], add=True)
        pltpu.async_copy(feat_hbm.at[src_v.at[j + 3]], buf1, sem1)
        return 0

    lax.fori_loop(0, _NCHUNK // 2 - 1, step, 0)
    wait_gather(buf0, sem0)
    pltpu.sync_copy(buf0, h_sh.at[dst_v.at[_NCHUNK - 2]], add=True)
    wait_gather(buf1, sem1)
    pltpu.sync_copy(buf1, h_sh.at[dst_v.at[_NCHUNK - 1]], add=True)

    plsc.subcore_barrier()
    pltpu.sync_copy(h_sh.at[pl.ds(r0, _RPT)],
                    out_hbm.at[c, pl.ds(r0, _RPT)])


def _lin_body(h_ref, w_ref, b_ref, o_ref):
    o_ref[...] = lax.dot_general(
        h_ref[0], w_ref[...], (((1,), (1,)), ((), ())),
        preferred_element_type=jnp.float32) + b_ref[...]


_linear = pl.pallas_call(
    _lin_body,
    grid=(10,),
    in_specs=[
        pl.BlockSpec((1, 1000, _D), lambda i: (i // 5, i % 5, 0)),
        pl.BlockSpec((_D, _D), lambda i: (0, 0)),
        pl.BlockSpec((1, _D), lambda i: (0, 0)),
    ],
    out_specs=pl.BlockSpec((1000, _D), lambda i: (i, 0)),
    out_shape=jax.ShapeDtypeStruct((_N, _D), jnp.float32),
)


@jax.jit
def kernel(features, edge_index, W, b):
    src = edge_index[0].reshape(_NS, _NCHUNK, _CH)
    dst = edge_index[1]
    halves = []
    for cc in range(_NC):
        lo = cc * _NH
        local = jnp.where((dst >= lo) & (dst < lo + _NH), dst - lo, _NH)
        halves.append(local.reshape(_NS, _NCHUNK, _CH))
    dst2 = jnp.stack(halves)
    zeros = jnp.zeros((_NP, _D), jnp.float32)
    hpart = _sc_gather_scatter(features, src, dst2, zeros)
    return _linear(hpart, W, b.reshape(1, _D))
